# Initial kernel scaffold; baseline (speedup 1.0000x reference)
import jax
import jax.numpy as jnp
from jax.experimental import pallas as pl


def kernel(queries, keys, values, attn_mask):
    raise NotImplementedError("write your pallas kernel here")


# speedup vs baseline: 1.8311x; 1.8311x over previous
import math

import jax
import jax.numpy as jnp
import numpy as np
from jax.experimental import pallas as pl
from jax.experimental.pallas import tpu as pltpu

_B, _L, _H, _D = 4, 2048, 16, 64
_U = 5 * int(np.ceil(np.log(_L)))
_TQ = 256
_TK = 256
_NEG = -1e30


def _count_matrix_T() -> np.ndarray:
    idx = np.asarray(
        jax.random.randint(jax.random.key(1234), (_L, _U), 0, _L)
    )
    c = np.zeros((_L, _L), dtype=np.float32)
    np.add.at(c, (np.arange(_L)[:, None], idx), 1.0)
    return np.ascontiguousarray(c.T)


_C_T_HOST = _count_matrix_T()


def _body(c_ref, q_ref, k_ref, v_ref, o_ref, m_scr):
    scale = 1.0 / math.sqrt(_D)

    def qb_body(qi, m_line):
        q_blk = q_ref[pl.ds(qi * _TQ, _TQ), :]

        def kb_body(ki, carry):
            rmax, rsum = carry
            k_blk = k_ref[pl.ds(ki * _TK, _TK), :]
            s = jax.lax.dot_general(
                k_blk, q_blk, (((1,), (1,)), ((), ())),
                preferred_element_type=jnp.float32,
            )
            c = c_ref[pl.ds(ki * _TK, _TK), pl.ds(qi * _TQ, _TQ)]
            masked = jnp.where(c > 0, s, _NEG)
            rmax = jnp.maximum(rmax, jnp.max(masked, axis=0, keepdims=True))
            rsum = rsum + jnp.sum(c * s, axis=0, keepdims=True)
            return rmax, rsum

        rmax, rsum = jax.lax.fori_loop(
            0, _L // _TK, kb_body,
            (jnp.full((1, _TQ), _NEG, jnp.float32),
             jnp.zeros((1, _TQ), jnp.float32)),
        )
        m_scr[:, pl.ds(qi * _TQ, _TQ)] = rmax - rsum * (1.0 / _L)
        return 0

    jax.lax.fori_loop(0, _L // _TQ, qb_body, 0)
    m_line = m_scr[:, :]

    qiota = jax.lax.broadcasted_iota(jnp.int32, (1, _L), 1)

    def top_body(i, carry):
        m, sel = carry
        mx = jnp.max(m)
        cand = jnp.where(m == mx, qiota, jnp.int32(_L))
        amin = jnp.min(cand)
        hit = qiota == amin
        sel = jnp.where(hit, i, sel)
        m = jnp.where(hit, _NEG, m)
        return m, sel

    _, sel = jax.lax.fori_loop(
        0, _U, top_body,
        (m_line, jnp.full((1, _L), -1, jnp.int32)),
    )

    riota = jax.lax.broadcasted_iota(jnp.int32, (_U, _L), 0)
    onehot = (riota == sel).astype(jnp.float32)

    q_red = jax.lax.dot_general(
        onehot, q_ref[:, :], (((1,), (0,)), ((), ())),
        preferred_element_type=jnp.float32,
    )
    scores = jax.lax.dot_general(
        q_red, k_ref[:, :], (((1,), (1,)), ((), ())),
        preferred_element_type=jnp.float32,
    ) * scale
    smax = jnp.max(scores, axis=1, keepdims=True)
    e = jnp.exp(scores - smax)
    attn = e / jnp.sum(e, axis=1, keepdims=True)
    upd = jax.lax.dot_general(
        attn, v_ref[:, :], (((1,), (0,)), ((), ())),
        preferred_element_type=jnp.float32,
    )

    vmean = jnp.mean(v_ref[:, :], axis=0, keepdims=True)
    ctx = jax.lax.dot_general(
        onehot, upd - vmean, (((0,), (0,)), ((), ())),
        preferred_element_type=jnp.float32,
    ) + vmean
    o_ref[:, :] = ctx


def kernel(queries, keys, values, attn_mask):
    del attn_mask
    B, L, H, D = queries.shape
    q = jnp.transpose(queries, (0, 2, 1, 3)).reshape(B * H, L, D)
    k = jnp.transpose(keys, (0, 2, 1, 3)).reshape(B * H, L, D)
    v = jnp.transpose(values, (0, 2, 1, 3)).reshape(B * H, L, D)
    c_t = jnp.asarray(_C_T_HOST)

    out = pl.pallas_call(
        _body,
        grid=(B * H,),
        in_specs=[
            pl.BlockSpec((_L, _L), lambda i: (0, 0)),
            pl.BlockSpec((None, _L, _D), lambda i: (i, 0, 0)),
            pl.BlockSpec((None, _L, _D), lambda i: (i, 0, 0)),
            pl.BlockSpec((None, _L, _D), lambda i: (i, 0, 0)),
        ],
        out_specs=pl.BlockSpec((None, _L, _D), lambda i: (i, 0, 0)),
        out_shape=jax.ShapeDtypeStruct((B * H, L, D), jnp.float32),
        scratch_shapes=[pltpu.VMEM((1, _L), jnp.float32)],
        compiler_params=pltpu.CompilerParams(
            dimension_semantics=("arbitrary",),
        ),
    )(c_t, q, k, v)
    return out.reshape(B, H, L, D)
